# Initial kernel scaffold; baseline (speedup 1.0000x reference)
import jax
import jax.numpy as jnp
from jax.experimental import pallas as pl


def kernel(coords, emb_W1, emb_b1, emb_W2, emb_b2, ln1_g, ln1_b, ln2_g, ln2_b, q_W, q_b, k_W, k_b, v_W, v_b, o_W, o_b, ff_W1, ff_b1, ff_W2, ff_b2, norm_g, norm_b, init_token, gru_Wih, gru_Whh, gru_bih, gru_bhh, qry_W, qry_b, key_W, key_b):
    raise NotImplementedError("write your pallas kernel here")


# speedup vs baseline: 1.3982x; 1.3982x over previous
import math

import jax
import jax.numpy as jnp
from jax import lax
from jax.experimental import pallas as pl

B, N, D, H, FF, L = 128, 256, 128, 8, 512, 3
DH = D // H
_SCALE = 1.0 / math.sqrt(D)
_CH = 32
_BF = jnp.bfloat16
_F32 = jnp.float32


def _bdot(a, b):
    return jnp.dot(a.astype(_BF), b.astype(_BF),
                   preferred_element_type=_F32)


def _ln_rows(x, g, b):
    xt = x.T
    m = jnp.mean(xt, axis=0, keepdims=True)
    v = jnp.mean((xt - m) ** 2, axis=0, keepdims=True)
    ht = (xt - m) / jnp.sqrt(v + 1e-5)
    return ht.T * g + b


def _head_masks():
    r_h = lax.broadcasted_iota(jnp.int32, (D, D), 0) // DH
    c_h = lax.broadcasted_iota(jnp.int32, (D, D), 1) // DH
    bd = (r_h == c_h).astype(_F32)
    eye = (lax.broadcasted_iota(jnp.int32, (D, D), 0)
           == lax.broadcasted_iota(jnp.int32, (D, D), 1)).astype(_F32)
    return bd, eye


def _full(shape):
    return pl.BlockSpec(shape, lambda i: (0,) * len(shape))


def _bnd(last):
    return pl.BlockSpec((1, N, last), lambda i: (i, 0, 0))


def _emb_front_body(coords_ref, ew1, eb1, ew2, eb2, l1g, l1b, qW, qb, kW,
                    kb, vW, vb, x_o, preq_o, prek_o, v_o):
    c = coords_ref[0].astype(_BF).astype(_F32)
    w1 = ew1[...].astype(_BF).astype(_F32)
    x = jax.nn.relu(c[:, 0:1] * w1[0:1, :] + c[:, 1:2] * w1[1:2, :]
                    + eb1[...])
    x = _bdot(x, ew2[...]) + eb2[...]
    x_o[0] = x
    h = _ln_rows(x, l1g[...], l1b[...])
    preq_o[0] = _bdot(h, qW[...]) + qb[...]
    prek_o[0] = _bdot(h, kW[...]) + kb[...]
    v_o[0] = _bdot(h, vW[...]) + vb[...]


def _mid_front_body(g_ref, x1_ref, fW2, fb2, l1g, l1b, qW, qb, kW, kb, vW,
                    vb, x_o, preq_o, prek_o, v_o):
    x = x1_ref[0] + (_bdot(g_ref[0], fW2[...]) + fb2[...])
    x_o[0] = x
    h = _ln_rows(x, l1g[...], l1b[...])
    preq_o[0] = _bdot(h, qW[...]) + qb[...]
    prek_o[0] = _bdot(h, kW[...]) + kb[...]
    v_o[0] = _bdot(h, vW[...]) + vb[...]


def _attn_body(q_ref, k_ref, v_ref, x_ref, oW, ob, l2g, l2b, fW1, fb1,
               x1_o, preg_o):
    q = q_ref[0]
    k = k_ref[0]
    v = v_ref[0]
    bd, eye = _head_masks()
    kv = lax.dot_general(k.astype(_BF), v.astype(_BF),
                         (((0,), (0,)), ((), ())),
                         preferred_element_type=_F32) * bd
    ksum = jnp.sum(k, axis=0, keepdims=True)
    ksb = ksum.astype(_BF).astype(_F32)
    m_ks = _bdot(eye * ksb, bd)
    den = _bdot(q, m_ks)
    z = 1.0 / (den + 1e-6)
    o = _bdot(q, kv) * z
    x1 = x_ref[0] + (_bdot(o, oW[...]) + ob[...])
    x1_o[0] = x1
    h2 = _ln_rows(x1, l2g[...], l2b[...])
    preg_o[0] = _bdot(h2, fW1[...]) + fb1[...]


def _final_body(g_ref, x1_ref, fW2, fb2, ng, nbias, kyW, kyb,
                enc_o, keys_o, st0_o):
    x = x1_ref[0] + (_bdot(g_ref[0], fW2[...]) + fb2[...])
    e = _ln_rows(x, ng[...], nbias[...])
    enc_o[0] = e
    keys_o[0] = (_bdot(e, kyW[...]) + kyb[...]).astype(_BF)
    st0_o[0] = jnp.mean(e, axis=0, keepdims=True)


def _dec_body(enc_ref, keys_ref, st0_ref, init_ref, wih_ref, whh_ref,
              bih_ref, bhh_ref, qw_ref, qb_ref,
              tours_ref, logp_ref, ent_ref):
    iota_n = lax.broadcasted_iota(jnp.int32, (B, N), 1)
    neg_inf = jnp.float32(-jnp.inf)

    def step(t, carry):
        state, prev, alivef, tacc, lacc, eacc = carry
        alive = alivef > 0.5
        gi = _bdot(prev, wih_ref[...]) + bih_ref[...]
        gh = _bdot(state, whh_ref[...]) + bhh_ref[...]
        r = jax.nn.sigmoid(gi[:, :D] + gh[:, :D])
        zz = jax.nn.sigmoid(gi[:, D:2 * D] + gh[:, D:2 * D])
        nn_ = jnp.tanh(gi[:, 2 * D:] + r * gh[:, 2 * D:])
        state = (1.0 - zz) * nn_ + zz * state
        query = _bdot(state, qw_ref[...]) + qb_ref[...]
        qf = query.astype(_BF).astype(_F32)

        parts = []
        for cs in range(0, N, _CH):
            kc = keys_ref[:, cs:cs + _CH, :].astype(_F32)
            parts.append(jnp.sum(kc * qf[:, None, :], axis=-1))
        logits = jnp.concatenate(parts, axis=1) * _SCALE

        masked = jnp.where(alive, logits, neg_inf)
        m = jnp.max(masked, axis=1, keepdims=True)
        s = masked - m
        logp = s - jnp.log(jnp.sum(jnp.exp(s), axis=1, keepdims=True))
        probs = jnp.exp(logp)
        ent = -jnp.sum(probs * jnp.where(alive, logp, 0.0), axis=1,
                       keepdims=True)
        pm = jnp.max(probs, axis=1, keepdims=True)
        cand = jnp.where(probs == pm, iota_n, N)
        idx = jnp.min(cand, axis=1, keepdims=True)
        onehot = iota_n == idx
        slp = jnp.sum(jnp.where(onehot, logp, 0.0), axis=1, keepdims=True)

        ohf = onehot.astype(_F32)
        prev_new = jnp.zeros((B, D), _F32)
        for cs in range(0, N, _CH):
            ec = enc_ref[:, cs:cs + _CH, :]
            prev_new = prev_new + jnp.sum(
                ec * ohf[:, cs:cs + _CH, None], axis=1)

        alivef = jnp.where(onehot, 0.0, alivef)
        at_t = iota_n == t
        tacc = jnp.where(at_t, idx, tacc)
        lacc = jnp.where(at_t, slp, lacc)
        eacc = jnp.where(at_t, ent, eacc)
        return state, prev_new, alivef, tacc, lacc, eacc

    carry0 = (st0_ref[...],
              jnp.broadcast_to(init_ref[...], (B, D)),
              jnp.ones((B, N), jnp.float32),
              jnp.zeros((B, N), jnp.int32),
              jnp.zeros((B, N), jnp.float32),
              jnp.zeros((B, N), jnp.float32))
    _, _, _, tacc, lacc, eacc = lax.fori_loop(0, N, step, carry0)
    tours_ref[...] = tacc
    logp_ref[...] = lacc
    ent_ref[...] = eacc


_BND_F32 = lambda last: jax.ShapeDtypeStruct((B, N, last), jnp.float32)


def _encode(coords, emb_W1, emb_b1, emb_W2, emb_b2, ln1_g, ln1_b, ln2_g,
            ln2_b, q_W, q_b, k_W, k_b, v_W, v_b, o_W, o_b, ff_W1, ff_b1,
            ff_W2, ff_b2, norm_g, norm_b, key_W, key_b):
    r1 = lambda a: a.reshape(1, -1)

    def front(l, g, x1):
        wargs = (ln1_g[l:l + 1], ln1_b[l:l + 1], q_W[l], r1(q_b[l]),
                 k_W[l], r1(k_b[l]), v_W[l], r1(v_b[l]))
        if l == 0:
            ops = (coords, emb_W1, r1(emb_b1), emb_W2, r1(emb_b2)) + wargs
            in_specs = [_bnd(2)] + [_full(o.shape) for o in ops[1:]]
            body = _emb_front_body
        else:
            ops = (g, x1, ff_W2[l - 1], r1(ff_b2[l - 1])) + wargs
            in_specs = [_bnd(FF), _bnd(D)] + [_full(o.shape)
                                              for o in ops[2:]]
            body = _mid_front_body
        return pl.pallas_call(
            body, grid=(B,), in_specs=in_specs,
            out_specs=[_bnd(D)] * 4,
            out_shape=[_BND_F32(D)] * 4,
        )(*ops)

    def attn(l, q, k, v, x):
        ops = (q, k, v, x, o_W[l], r1(o_b[l]), ln2_g[l:l + 1],
               ln2_b[l:l + 1], ff_W1[l], r1(ff_b1[l]))
        return pl.pallas_call(
            _attn_body, grid=(B,),
            in_specs=[_bnd(D)] * 4 + [_full(o.shape) for o in ops[4:]],
            out_specs=[_bnd(D), _bnd(FF)],
            out_shape=[_BND_F32(D), _BND_F32(FF)],
        )(*ops)

    g = x1 = None
    for l in range(L):
        x, preq, prek, v = front(l, g, x1)
        q = jax.nn.elu(preq) + 1.0
        k = jax.nn.elu(prek) + 1.0
        x1, preg = attn(l, q, k, v, x)
        g = jax.nn.gelu(preg, approximate=False)

    ops = (g, x1, ff_W2[L - 1], r1(ff_b2[L - 1]), r1(norm_g), r1(norm_b),
           key_W, r1(key_b))
    enc, keys, st0 = pl.pallas_call(
        _final_body, grid=(B,),
        in_specs=[_bnd(FF), _bnd(D)] + [_full(o.shape) for o in ops[2:]],
        out_specs=[_bnd(D), _bnd(D), pl.BlockSpec((1, 1, D),
                                                  lambda i: (i, 0, 0))],
        out_shape=[_BND_F32(D),
                   jax.ShapeDtypeStruct((B, N, D), jnp.bfloat16),
                   jax.ShapeDtypeStruct((B, 1, D), jnp.float32)],
    )(*ops)
    return enc, keys, st0


def _decode(enc, keys, state0, init_token, gru_Wih, gru_Whh, gru_bih,
            gru_bhh, qry_W, qry_b):
    r1 = lambda a: a.reshape(1, -1)
    return pl.pallas_call(
        _dec_body,
        out_shape=[
            jax.ShapeDtypeStruct((B, N), jnp.int32),
            jax.ShapeDtypeStruct((B, N), jnp.float32),
            jax.ShapeDtypeStruct((B, N), jnp.float32),
        ],
    )(enc, keys, state0, r1(init_token), gru_Wih.T, gru_Whh.T,
      r1(gru_bih), r1(gru_bhh), qry_W, r1(qry_b))


def kernel(coords, emb_W1, emb_b1, emb_W2, emb_b2, ln1_g, ln1_b, ln2_g,
           ln2_b, q_W, q_b, k_W, k_b, v_W, v_b, o_W, o_b, ff_W1, ff_b1,
           ff_W2, ff_b2, norm_g, norm_b, init_token, gru_Wih, gru_Whh,
           gru_bih, gru_bhh, qry_W, qry_b, key_W, key_b):
    enc, keys, state0 = _encode(
        coords, emb_W1, emb_b1, emb_W2, emb_b2, ln1_g, ln1_b, ln2_g,
        ln2_b, q_W, q_b, k_W, k_b, v_W, v_b, o_W, o_b, ff_W1, ff_b1,
        ff_W2, ff_b2, norm_g, norm_b, key_W, key_b)
    return _decode(enc, keys, state0.reshape(B, D), init_token, gru_Wih,
                   gru_Whh, gru_bih, gru_bhh, qry_W, qry_b)
